# baseline (speedup 1.0000x reference)
import jax
import jax.numpy as jnp
from jax import lax
from jax.experimental import pallas as pl
from jax.experimental.pallas import tpu as pltpu
from jax.experimental.pallas import tpu_sc as plsc

N = 1048576
K_RANK = 996146
NS = 16
E = N // NS
VI = E // 16
NBINS = 256


def _to_key(b):
  m = lax.shift_right_arithmetic(b, jnp.broadcast_to(jnp.int32(31), b.shape))
  return b ^ (m & jnp.int32(0x7FFFFFFF))


def _es_body(losses_hbm, out_hbm,
             data_v, hist_v, tot_v, all_v,
             statf_v, stati_v, allf_v, alli_v, out_v,
             comb_sh, statf_sh, stati_sh):
  cid = lax.axis_index("c")
  sid = lax.axis_index("s")

  lanes = lax.iota(jnp.int32, 16)
  ones = jnp.ones((16,), jnp.int32)
  zeros16 = jnp.zeros((16,), jnp.int32)

  pltpu.sync_copy(losses_hbm.at[pl.ds(sid * E, E)], data_v)

  def zero_hist(j, _):
    hist_v[pl.ds(j * 16, 16)] = zeros16
    return 0
  lax.fori_loop(0, NBINS, zero_hist, 0)

  key_prefix = jnp.int32(0)
  rank = jnp.int32(K_RANK)
  gt_cnt = jnp.int32(0)
  eq_cnt = jnp.int32(0)

  for p in range(4):
    shift = 24 - 8 * p

    if p == 0:
      @plsc.parallel_loop(0, VI, unroll=8)
      def _(i):
        x = data_v[pl.ds(i * 16, 16)]
        key = _to_key(lax.bitcast_convert_type(x, jnp.int32))
        data_v[pl.ds(i * 16, 16)] = lax.bitcast_convert_type(key, jnp.float32)
        digit = ((key >> 24) & 255) ^ 128
        plsc.addupdate_scatter(hist_v, [digit * 16 + lanes], ones)
    else:
      himask = jnp.int32(-(1 << (shift + 8)))
      pref = key_prefix

      @plsc.parallel_loop(0, VI, unroll=8)
      def _(i):
        key = lax.bitcast_convert_type(data_v[pl.ds(i * 16, 16)], jnp.int32)
        match = (key & himask) == pref
        digit = (key >> shift) & 255
        plsc.addupdate_scatter(hist_v, [digit * 16 + lanes], ones, mask=match)

    @plsc.parallel_loop(0, 16, unroll=2)
    def _(dc):
      acc = zeros16
      for j in range(16):
        v = hist_v[pl.ds((dc * 16 + j) * 16, 16)]
        for m in (8, 4, 2, 1):
          v = v + v[lanes ^ m]
        acc = jnp.where(lanes == j, v, acc)
      tot_v[pl.ds(dc * 16, 16)] = acc

    if p < 3:
      @plsc.parallel_loop(0, NBINS, unroll=8)
      def _(j):
        hist_v[pl.ds(j * 16, 16)] = zeros16

    pltpu.sync_copy(tot_v, comb_sh.at[pl.ds((p * NS + sid) * NBINS, NBINS)])
    plsc.subcore_barrier()
    pltpu.sync_copy(comb_sh.at[pl.ds(p * NS * NBINS, NS * NBINS)], all_v)

    def scan(dc, carry):
      base, dselv, belowv, cselv = carry
      acc = all_v[pl.ds(dc * 16, 16)]
      for t in range(1, NS):
        acc = acc + all_v[pl.ds(t * NBINS + dc * 16, 16)]
      inc = plsc.cumsum(acc)
      cumb = (base + inc) - acc
      take = jnp.logical_and(cumb <= rank, cumb + acc > rank)
      digits = dc * 16 + lanes
      dselv = jnp.maximum(dselv, jnp.where(take, digits, jnp.int32(-1)))
      belowv = jnp.maximum(belowv, jnp.where(take, cumb, jnp.int32(-1)))
      cselv = jnp.maximum(cselv, jnp.where(take, acc, jnp.int32(-1)))
      return base + jnp.sum(acc), dselv, belowv, cselv

    minus1 = jnp.full((16,), -1, jnp.int32)
    class_total, dselv, belowv, cselv = lax.fori_loop(
        0, 16, scan, (jnp.int32(0), minus1, minus1, minus1))
    dsel = jnp.max(dselv)
    below = jnp.max(belowv)
    c_sel = jnp.max(cselv)

    gt_cnt = gt_cnt + (class_total - below - c_sel)
    eq_cnt = c_sel
    rank = rank - below
    if p == 0:
      key_prefix = key_prefix | ((dsel ^ 128) << 24)
    else:
      key_prefix = key_prefix | (dsel << shift)

  vk_vec = lax.bitcast_convert_type(
      _to_key(jnp.full((16,), key_prefix, jnp.int32)), jnp.float32)

  kk_vec = jnp.full((16,), key_prefix, jnp.int32)

  def stats(i, carry):
    acc_sum, acc_min = carry
    key = lax.bitcast_convert_type(data_v[pl.ds(i * 16, 16)], jnp.int32)
    x = lax.bitcast_convert_type(_to_key(key), jnp.float32)
    gt = key > kk_vec
    acc_sum = acc_sum + jnp.where(gt, x, jnp.float32(0.0))
    acc_min = jnp.minimum(acc_min, jnp.where(gt, key, jnp.int32(0x7FFFFFFF)))
    return acc_sum, acc_min

  acc_sum, acc_min = plsc.parallel_loop(
      0, VI, unroll=8,
      carry=(jnp.zeros((16,), jnp.float32),
             jnp.full((16,), 0x7FFFFFFF, jnp.int32)))(
                 lambda i, c: stats(i, c))

  statf_v[...] = jnp.where(lanes == 0, jnp.sum(acc_sum), jnp.float32(0.0))
  stati_v[...] = jnp.where(lanes == 0, jnp.min(acc_min),
                           jnp.int32(0x7FFFFFFF))
  pltpu.sync_copy(statf_v, statf_sh.at[pl.ds(sid * 16, 16)])
  pltpu.sync_copy(stati_v, stati_sh.at[pl.ds(sid * 16, 16)])
  plsc.subcore_barrier()

  @pl.when(jnp.logical_and(cid == 0, sid == 0))
  def _():
    pltpu.sync_copy(statf_sh, allf_v)
    pltpu.sync_copy(stati_sh, alli_v)
    accf = jnp.zeros((16,), jnp.float32)
    acci = jnp.full((16,), 0x7FFFFFFF, jnp.int32)
    for t in range(NS):
      accf = accf + allf_v[pl.ds(t * 16, 16)]
      acci = jnp.minimum(acci, alli_v[pl.ds(t * 16, 16)])
    sum_gt = accf[0]
    cnt_gt = gt_cnt.astype(jnp.float32)
    cnt_eq = eq_cnt.astype(jnp.float32)
    min_gt = acci[0]

    vnext_vec = lax.bitcast_convert_type(
        _to_key(jnp.full((16,), min_gt, jnp.int32)), jnp.float32)
    cnt_le = jnp.float32(N) - cnt_gt
    have_same = jnp.full((16,), cnt_le >= jnp.float32(K_RANK + 2))
    vnext_vec = jnp.where(have_same, vk_vec, vnext_vec)

    var_vec = vk_vec * jnp.float32(0.75) + vnext_vec * jnp.float32(0.25)
    inc_eq = var_vec <= vk_vec
    tail_sum = jnp.where(inc_eq, sum_gt + cnt_eq * vk_vec,
                         jnp.full((16,), sum_gt))
    tail_cnt = jnp.where(inc_eq, cnt_gt + cnt_eq,
                         jnp.full((16,), cnt_gt))
    out_v[...] = tail_sum / tail_cnt
    pltpu.sync_copy(out_v, out_hbm)


@jax.jit
def _es_pallas(losses):
  mesh = plsc.VectorSubcoreMesh(core_axis_name="c", subcore_axis_name="s")
  f = pl.kernel(
      _es_body,
      out_type=jax.ShapeDtypeStruct((16,), jnp.float32),
      mesh=mesh,
      compiler_params=pltpu.CompilerParams(needs_layout_passes=False),
      scratch_types=[
          pltpu.VMEM((E,), jnp.float32),
          pltpu.VMEM((NBINS * 16,), jnp.int32),
          pltpu.VMEM((NBINS,), jnp.int32),
          pltpu.VMEM((NS * NBINS,), jnp.int32),
          pltpu.VMEM((16,), jnp.float32),
          pltpu.VMEM((16,), jnp.int32),
          pltpu.VMEM((NS * 16,), jnp.float32),
          pltpu.VMEM((NS * 16,), jnp.int32),
          pltpu.VMEM((16,), jnp.float32),
          pltpu.VMEM_SHARED((4 * NS * NBINS,), jnp.int32),
          pltpu.VMEM_SHARED((NS * 16,), jnp.float32),
          pltpu.VMEM_SHARED((NS * 16,), jnp.int32),
      ],
  )
  return f(losses)


def kernel(losses):
  return _es_pallas(losses)[0]


# speedup vs baseline: 1.0445x; 1.0445x over previous
import jax
import jax.numpy as jnp
from jax import lax
from jax.experimental import pallas as pl
from jax.experimental.pallas import tpu as pltpu
from jax.experimental.pallas import tpu_sc as plsc

N = 1048576
K_RANK = 996146
NS = 16
E = N // NS
VI = E // 16
NBINS = 256


def _to_key(b):
  m = lax.shift_right_arithmetic(b, jnp.broadcast_to(jnp.int32(31), b.shape))
  return b ^ (m & jnp.int32(0x7FFFFFFF))


def _es_body(losses_hbm, out_hbm,
             data_v, hist_v, tot_v, all_v,
             statf_v, stati_v, allf_v, alli_v, out_v,
             comb_sh, statf_sh, stati_sh):
  cid = lax.axis_index("c")
  sid = lax.axis_index("s")

  lanes = lax.iota(jnp.int32, 16)
  ones = jnp.ones((16,), jnp.int32)
  zeros16 = jnp.zeros((16,), jnp.int32)

  pltpu.sync_copy(losses_hbm.at[pl.ds(sid * E, E)], data_v)

  def zero_hist(j, _):
    hist_v[pl.ds(j * 16, 16)] = zeros16
    return 0
  lax.fori_loop(0, NBINS, zero_hist, 0)

  key_prefix = jnp.int32(0)
  rank = jnp.int32(K_RANK)
  gt_cnt = jnp.int32(0)
  eq_cnt = jnp.int32(0)

  for p in range(4):
    shift = 24 - 8 * p

    if p == 0:
      @plsc.parallel_loop(0, VI, unroll=8)
      def _(i):
        x = data_v[pl.ds(i * 16, 16)]
        key = _to_key(lax.bitcast_convert_type(x, jnp.int32))
        data_v[pl.ds(i * 16, 16)] = lax.bitcast_convert_type(key, jnp.float32)
        digit = ((key >> 24) & 255) ^ 128
        plsc.addupdate_scatter(hist_v, [digit * 16 + lanes], ones)
    else:
      himask = jnp.int32(-(1 << (shift + 8)))
      pref = key_prefix

      @plsc.parallel_loop(0, VI, unroll=8)
      def _(i):
        key = lax.bitcast_convert_type(data_v[pl.ds(i * 16, 16)], jnp.int32)
        match = (key & himask) == pref
        digit = (key >> shift) & 255
        plsc.addupdate_scatter(hist_v, [digit * 16 + lanes], ones, mask=match)

    def lane_reduce(dc, _):
      acc = zeros16
      for j in range(16):
        v = hist_v[pl.ds((dc * 16 + j) * 16, 16)]
        hist_v[pl.ds((dc * 16 + j) * 16, 16)] = zeros16
        acc = jnp.where(lanes == j, jnp.sum(v), acc)
      tot_v[pl.ds(dc * 16, 16)] = acc
      return 0
    lax.fori_loop(0, 16, lane_reduce, 0)

    pltpu.sync_copy(tot_v, comb_sh.at[pl.ds((p * NS + sid) * NBINS, NBINS)])
    plsc.subcore_barrier()
    pltpu.sync_copy(comb_sh.at[pl.ds(p * NS * NBINS, NS * NBINS)], all_v)

    def scan(dc, carry):
      base, dselv, belowv, cselv = carry
      acc = all_v[pl.ds(dc * 16, 16)]
      for t in range(1, NS):
        acc = acc + all_v[pl.ds(t * NBINS + dc * 16, 16)]
      inc = plsc.cumsum(acc)
      cumb = (base + inc) - acc
      take = jnp.logical_and(cumb <= rank, cumb + acc > rank)
      digits = dc * 16 + lanes
      dselv = jnp.maximum(dselv, jnp.where(take, digits, jnp.int32(-1)))
      belowv = jnp.maximum(belowv, jnp.where(take, cumb, jnp.int32(-1)))
      cselv = jnp.maximum(cselv, jnp.where(take, acc, jnp.int32(-1)))
      return base + jnp.sum(acc), dselv, belowv, cselv

    minus1 = jnp.full((16,), -1, jnp.int32)
    class_total, dselv, belowv, cselv = lax.fori_loop(
        0, 16, scan, (jnp.int32(0), minus1, minus1, minus1))
    dsel = jnp.max(dselv)
    below = jnp.max(belowv)
    c_sel = jnp.max(cselv)

    gt_cnt = gt_cnt + (class_total - below - c_sel)
    eq_cnt = c_sel
    rank = rank - below
    if p == 0:
      key_prefix = key_prefix | ((dsel ^ 128) << 24)
    else:
      key_prefix = key_prefix | (dsel << shift)

  vk_vec = lax.bitcast_convert_type(
      _to_key(jnp.full((16,), key_prefix, jnp.int32)), jnp.float32)

  kk_vec = jnp.full((16,), key_prefix, jnp.int32)

  def stats(i, carry):
    acc_sum, acc_min = carry
    key = lax.bitcast_convert_type(data_v[pl.ds(i * 16, 16)], jnp.int32)
    x = lax.bitcast_convert_type(_to_key(key), jnp.float32)
    gt = key > kk_vec
    acc_sum = acc_sum + jnp.where(gt, x, jnp.float32(0.0))
    acc_min = jnp.minimum(acc_min, jnp.where(gt, key, jnp.int32(0x7FFFFFFF)))
    return acc_sum, acc_min

  acc_sum, acc_min = plsc.parallel_loop(
      0, VI, unroll=8,
      carry=(jnp.zeros((16,), jnp.float32),
             jnp.full((16,), 0x7FFFFFFF, jnp.int32)))(
                 lambda i, c: stats(i, c))

  statf_v[...] = jnp.where(lanes == 0, jnp.sum(acc_sum), jnp.float32(0.0))
  stati_v[...] = jnp.where(lanes == 0, jnp.min(acc_min),
                           jnp.int32(0x7FFFFFFF))
  pltpu.sync_copy(statf_v, statf_sh.at[pl.ds(sid * 16, 16)])
  pltpu.sync_copy(stati_v, stati_sh.at[pl.ds(sid * 16, 16)])
  plsc.subcore_barrier()

  @pl.when(jnp.logical_and(cid == 0, sid == 0))
  def _():
    pltpu.sync_copy(statf_sh, allf_v)
    pltpu.sync_copy(stati_sh, alli_v)
    accf = jnp.zeros((16,), jnp.float32)
    acci = jnp.full((16,), 0x7FFFFFFF, jnp.int32)
    for t in range(NS):
      accf = accf + allf_v[pl.ds(t * 16, 16)]
      acci = jnp.minimum(acci, alli_v[pl.ds(t * 16, 16)])
    sum_gt = accf[0]
    cnt_gt = gt_cnt.astype(jnp.float32)
    cnt_eq = eq_cnt.astype(jnp.float32)
    min_gt = acci[0]

    vnext_vec = lax.bitcast_convert_type(
        _to_key(jnp.full((16,), min_gt, jnp.int32)), jnp.float32)
    cnt_le = jnp.float32(N) - cnt_gt
    have_same = jnp.full((16,), cnt_le >= jnp.float32(K_RANK + 2))
    vnext_vec = jnp.where(have_same, vk_vec, vnext_vec)

    var_vec = vk_vec * jnp.float32(0.75) + vnext_vec * jnp.float32(0.25)
    inc_eq = var_vec <= vk_vec
    tail_sum = jnp.where(inc_eq, sum_gt + cnt_eq * vk_vec,
                         jnp.full((16,), sum_gt))
    tail_cnt = jnp.where(inc_eq, cnt_gt + cnt_eq,
                         jnp.full((16,), cnt_gt))
    out_v[...] = tail_sum / tail_cnt
    pltpu.sync_copy(out_v, out_hbm)


@jax.jit
def _es_pallas(losses):
  mesh = plsc.VectorSubcoreMesh(core_axis_name="c", subcore_axis_name="s")
  f = pl.kernel(
      _es_body,
      out_type=jax.ShapeDtypeStruct((16,), jnp.float32),
      mesh=mesh,
      compiler_params=pltpu.CompilerParams(needs_layout_passes=False),
      scratch_types=[
          pltpu.VMEM((E,), jnp.float32),
          pltpu.VMEM((NBINS * 16,), jnp.int32),
          pltpu.VMEM((NBINS,), jnp.int32),
          pltpu.VMEM((NS * NBINS,), jnp.int32),
          pltpu.VMEM((16,), jnp.float32),
          pltpu.VMEM((16,), jnp.int32),
          pltpu.VMEM((NS * 16,), jnp.float32),
          pltpu.VMEM((NS * 16,), jnp.int32),
          pltpu.VMEM((16,), jnp.float32),
          pltpu.VMEM_SHARED((4 * NS * NBINS,), jnp.int32),
          pltpu.VMEM_SHARED((NS * 16,), jnp.float32),
          pltpu.VMEM_SHARED((NS * 16,), jnp.int32),
      ],
  )
  return f(losses)


def kernel(losses):
  return _es_pallas(losses)[0]


# speedup vs baseline: 1.0618x; 1.0166x over previous
import jax
import jax.numpy as jnp
from jax import lax
from jax.experimental import pallas as pl
from jax.experimental.pallas import tpu as pltpu
from jax.experimental.pallas import tpu_sc as plsc

N = 1048576
K_RANK = 996146
NS = 16
E = N // NS
VI = E // 16
NBINS = 256


def _to_key(b):
  m = lax.shift_right_arithmetic(b, jnp.broadcast_to(jnp.int32(31), b.shape))
  return b ^ (m & jnp.int32(0x7FFFFFFF))


def _es_body(losses_hbm, out_hbm,
             data_v, hist_v, tot_v, all_v,
             statf_v, stati_v, allf_v, alli_v, out_v,
             comb_sh, statf_sh, stati_sh, dma_sem):
  cid = lax.axis_index("c")
  sid = lax.axis_index("s")

  lanes = lax.iota(jnp.int32, 16)
  ones = jnp.ones((16,), jnp.int32)
  zeros16 = jnp.zeros((16,), jnp.int32)

  cp = pltpu.make_async_copy(losses_hbm.at[pl.ds(sid * E, E)], data_v, dma_sem)
  cp.start()

  @plsc.parallel_loop(0, NBINS, unroll=8)
  def _(j):
    hist_v[pl.ds(j * 16, 16)] = zeros16

  cp.wait()

  key_prefix = jnp.int32(0)
  rank = jnp.int32(K_RANK)
  gt_cnt = jnp.int32(0)
  eq_cnt = jnp.int32(0)

  for p in range(4):
    shift = 24 - 8 * p

    if p == 0:
      @plsc.parallel_loop(0, VI, unroll=8)
      def _(i):
        x = data_v[pl.ds(i * 16, 16)]
        key = _to_key(lax.bitcast_convert_type(x, jnp.int32))
        data_v[pl.ds(i * 16, 16)] = lax.bitcast_convert_type(key, jnp.float32)
        digit = ((key >> 24) & 255) ^ 128
        plsc.addupdate_scatter(hist_v, [digit * 16 + lanes], ones)
    else:
      himask = jnp.int32(-(1 << (shift + 8)))
      pref = key_prefix

      @plsc.parallel_loop(0, VI, unroll=8)
      def _(i):
        key = lax.bitcast_convert_type(data_v[pl.ds(i * 16, 16)], jnp.int32)
        match = (key & himask) == pref
        digit = (key >> shift) & 255
        plsc.addupdate_scatter(hist_v, [digit * 16 + lanes], ones, mask=match)

    def lane_reduce(dc, _):
      acc = zeros16
      for j in range(16):
        v = hist_v[pl.ds((dc * 16 + j) * 16, 16)]
        hist_v[pl.ds((dc * 16 + j) * 16, 16)] = zeros16
        acc = jnp.where(lanes == j, jnp.sum(v), acc)
      tot_v[pl.ds(dc * 16, 16)] = acc
      return 0
    lax.fori_loop(0, 16, lane_reduce, 0)

    pltpu.sync_copy(tot_v, comb_sh.at[pl.ds((p * NS + sid) * NBINS, NBINS)])
    plsc.subcore_barrier()
    pltpu.sync_copy(comb_sh.at[pl.ds(p * NS * NBINS, NS * NBINS)], all_v)

    def scan(dc, carry):
      base, dselv, belowv, cselv = carry
      acc = all_v[pl.ds(dc * 16, 16)]
      for t in range(1, NS):
        acc = acc + all_v[pl.ds(t * NBINS + dc * 16, 16)]
      inc = plsc.cumsum(acc)
      cumb = (base + inc) - acc
      take = jnp.logical_and(cumb <= rank, cumb + acc > rank)
      digits = dc * 16 + lanes
      dselv = jnp.maximum(dselv, jnp.where(take, digits, jnp.int32(-1)))
      belowv = jnp.maximum(belowv, jnp.where(take, cumb, jnp.int32(-1)))
      cselv = jnp.maximum(cselv, jnp.where(take, acc, jnp.int32(-1)))
      return base + jnp.sum(acc), dselv, belowv, cselv

    minus1 = jnp.full((16,), -1, jnp.int32)
    class_total, dselv, belowv, cselv = lax.fori_loop(
        0, 16, scan, (jnp.int32(0), minus1, minus1, minus1))
    dsel = jnp.max(dselv)
    below = jnp.max(belowv)
    c_sel = jnp.max(cselv)

    gt_cnt = gt_cnt + (class_total - below - c_sel)
    eq_cnt = c_sel
    rank = rank - below
    if p == 0:
      key_prefix = key_prefix | ((dsel ^ 128) << 24)
    else:
      key_prefix = key_prefix | (dsel << shift)

  vk_vec = lax.bitcast_convert_type(
      _to_key(jnp.full((16,), key_prefix, jnp.int32)), jnp.float32)

  kk_vec = jnp.full((16,), key_prefix, jnp.int32)

  def stats(i, carry):
    acc_sum, acc_min = carry
    key = lax.bitcast_convert_type(data_v[pl.ds(i * 16, 16)], jnp.int32)
    x = lax.bitcast_convert_type(_to_key(key), jnp.float32)
    gt = key > kk_vec
    acc_sum = acc_sum + jnp.where(gt, x, jnp.float32(0.0))
    acc_min = jnp.minimum(acc_min, jnp.where(gt, key, jnp.int32(0x7FFFFFFF)))
    return acc_sum, acc_min

  acc_sum, acc_min = plsc.parallel_loop(
      0, VI, unroll=8,
      carry=(jnp.zeros((16,), jnp.float32),
             jnp.full((16,), 0x7FFFFFFF, jnp.int32)))(
                 lambda i, c: stats(i, c))

  statf_v[...] = jnp.where(lanes == 0, jnp.sum(acc_sum), jnp.float32(0.0))
  stati_v[...] = jnp.where(lanes == 0, jnp.min(acc_min),
                           jnp.int32(0x7FFFFFFF))
  pltpu.sync_copy(statf_v, statf_sh.at[pl.ds(sid * 16, 16)])
  pltpu.sync_copy(stati_v, stati_sh.at[pl.ds(sid * 16, 16)])
  plsc.subcore_barrier()

  @pl.when(jnp.logical_and(cid == 0, sid == 0))
  def _():
    pltpu.sync_copy(statf_sh, allf_v)
    pltpu.sync_copy(stati_sh, alli_v)
    accf = jnp.zeros((16,), jnp.float32)
    acci = jnp.full((16,), 0x7FFFFFFF, jnp.int32)
    for t in range(NS):
      accf = accf + allf_v[pl.ds(t * 16, 16)]
      acci = jnp.minimum(acci, alli_v[pl.ds(t * 16, 16)])
    sum_gt = accf[0]
    cnt_gt = gt_cnt.astype(jnp.float32)
    cnt_eq = eq_cnt.astype(jnp.float32)
    min_gt = acci[0]

    vnext_vec = lax.bitcast_convert_type(
        _to_key(jnp.full((16,), min_gt, jnp.int32)), jnp.float32)
    cnt_le = jnp.float32(N) - cnt_gt
    have_same = jnp.full((16,), cnt_le >= jnp.float32(K_RANK + 2))
    vnext_vec = jnp.where(have_same, vk_vec, vnext_vec)

    var_vec = vk_vec * jnp.float32(0.75) + vnext_vec * jnp.float32(0.25)
    inc_eq = var_vec <= vk_vec
    tail_sum = jnp.where(inc_eq, sum_gt + cnt_eq * vk_vec,
                         jnp.full((16,), sum_gt))
    tail_cnt = jnp.where(inc_eq, cnt_gt + cnt_eq,
                         jnp.full((16,), cnt_gt))
    out_v[...] = tail_sum / tail_cnt
    pltpu.sync_copy(out_v, out_hbm)


@jax.jit
def _es_pallas(losses):
  mesh = plsc.VectorSubcoreMesh(core_axis_name="c", subcore_axis_name="s")
  f = pl.kernel(
      _es_body,
      out_type=jax.ShapeDtypeStruct((16,), jnp.float32),
      mesh=mesh,
      compiler_params=pltpu.CompilerParams(needs_layout_passes=False),
      scratch_types=[
          pltpu.VMEM((E,), jnp.float32),
          pltpu.VMEM((NBINS * 16,), jnp.int32),
          pltpu.VMEM((NBINS,), jnp.int32),
          pltpu.VMEM((NS * NBINS,), jnp.int32),
          pltpu.VMEM((16,), jnp.float32),
          pltpu.VMEM((16,), jnp.int32),
          pltpu.VMEM((NS * 16,), jnp.float32),
          pltpu.VMEM((NS * 16,), jnp.int32),
          pltpu.VMEM((16,), jnp.float32),
          pltpu.VMEM_SHARED((4 * NS * NBINS,), jnp.int32),
          pltpu.VMEM_SHARED((NS * 16,), jnp.float32),
          pltpu.VMEM_SHARED((NS * 16,), jnp.int32),
          pltpu.SemaphoreType.DMA,
      ],
  )
  return f(losses)


def kernel(losses):
  return _es_pallas(losses)[0]
